# baseline (speedup 1.0000x reference)
import functools

import jax
import jax.numpy as jnp
from jax import lax
from jax.experimental import pallas as pl
from jax.experimental.pallas import tpu as pltpu
from jax.experimental.pallas import tpu_sc as plsc

BATCH = 16384
EMB = 64
_info = plsc.get_sparse_core_info()
NC, NS, L = _info.num_cores, _info.num_subcores, _info.num_lanes
NW = NC * NS
BPW = BATCH // NW
CHUNK = 128
NCHUNK = BPW // CHUNK


def _make_sc_kernel():
  mesh = plsc.VectorSubcoreMesh(core_axis_name="c", subcore_axis_name="s")

  @functools.partial(
      pl.kernel,
      mesh=mesh,
      compiler_params=pltpu.CompilerParams(
          needs_layout_passes=False, use_tc_tiling_on_sc=False),
      out_type=jax.ShapeDtypeStruct((BATCH,), jnp.float32),
      scratch_types=[
          pltpu.VMEM((NCHUNK, CHUNK), jnp.int32),
          pltpu.VMEM((NCHUNK, CHUNK), jnp.int32),
          pltpu.VMEM((BPW, EMB), jnp.float32),
          pltpu.VMEM((BPW, EMB), jnp.float32),
          pltpu.VMEM((BPW,), jnp.float32),
          pltpu.SemaphoreType.DMA,
          pltpu.SemaphoreType.DMA,
      ],
  )
  def k(u_hbm, v_hbm, ue_hbm, ve_hbm, out_hbm,
        u_idx, v_idx, u_rows, v_rows, out_v, sem_u, sem_v):
    wid = lax.axis_index("s") * NC + lax.axis_index("c")
    base = wid * BPW

    pltpu.sync_copy(u_hbm.at[pl.ds(wid * NCHUNK, NCHUNK)], u_idx)
    pltpu.sync_copy(v_hbm.at[pl.ds(wid * NCHUNK, NCHUNK)], v_idx)

    copies = []
    for j in range(NCHUNK):
      dst = u_rows.at[pl.ds(j * CHUNK, CHUNK)]
      copies.append(pltpu.async_copy(ue_hbm.at[u_idx.at[j]], dst, sem_u))
      dst = v_rows.at[pl.ds(j * CHUNK, CHUNK)]
      copies.append(pltpu.async_copy(ve_hbm.at[v_idx.at[j]], dst, sem_v))
    for c in copies:
      c.wait()

    def body(g, _):
      rows = g * L + lax.iota(jnp.int32, L)
      acc = jnp.zeros((L,), jnp.float32)
      for d in range(EMB):
        col = jnp.full((L,), d, jnp.int32)
        a = plsc.load_gather(u_rows, [rows, col])
        b = plsc.load_gather(v_rows, [rows, col])
        acc = acc + a * b
      out_v[pl.ds(g * L, L)] = acc
      return _

    lax.fori_loop(0, BPW // L, body, None)

    pltpu.sync_copy(out_v, out_hbm.at[pl.ds(base, BPW)])

  return k


_sc_kernel = _make_sc_kernel()


def kernel(u, v, user_emb, item_emb):
  u2 = u.astype(jnp.int32).reshape(NW * NCHUNK, CHUNK)
  v2 = v.astype(jnp.int32).reshape(NW * NCHUNK, CHUNK)
  return _sc_kernel(u2, v2, user_emb, item_emb)
